# baseline (speedup 1.0000x reference)
import jax
import jax.numpy as jnp
from jax.experimental import pallas as pl
from jax.experimental.pallas import tpu as pltpu

_C = 81
_BLK = 4000
_NBUF = 8


def _body(lab_ref, out_ref, buf, sems):
    i = pl.program_id(0)
    grid = pl.num_programs(0)
    bpb = 100000 // _BLK
    s = jax.lax.rem(i, _NBUF)

    @pl.when(i >= _NBUF)
    def _():
        j = i - _NBUF
        pltpu.make_async_copy(
            buf.at[jax.lax.rem(j, _NBUF)],
            out_ref.at[j // bpb, pl.ds(jax.lax.rem(j, bpb) * _BLK, _BLK)],
            sems.at[jax.lax.rem(j, _NBUF)]).wait()

    lf = lab_ref[0].astype(jnp.float32)
    ones = jnp.ones((1, _C), jnp.float32)
    lb = jax.lax.dot_general(
        lf, ones, (((0,), (0,)), ((), ())),
        preferred_element_type=jnp.float32)
    iota = jax.lax.broadcasted_iota(
        jnp.int32, (_BLK, _C), 1).astype(jnp.float32)
    buf[s] = jnp.where(lb == iota, 1,
                       jnp.where(lb == -iota, -1, 0)).astype(jnp.int32)

    pltpu.make_async_copy(
        buf.at[s],
        out_ref.at[i // bpb, pl.ds(jax.lax.rem(i, bpb) * _BLK, _BLK)],
        sems.at[s]).start()

    @pl.when(i == grid - 1)
    def _():
        for d in range(_NBUF - 1):
            j0 = grid - _NBUF + d
            pltpu.make_async_copy(
                buf.at[jax.lax.rem(j0, _NBUF)],
                out_ref.at[j0 // bpb,
                           pl.ds(jax.lax.rem(j0, bpb) * _BLK, _BLK)],
                sems.at[jax.lax.rem(j0, _NBUF)]).wait()
        pltpu.make_async_copy(
            buf.at[s],
            out_ref.at[i // bpb, pl.ds(jax.lax.rem(i, bpb) * _BLK, _BLK)],
            sems.at[s]).wait()


def kernel(cls_label):
    b, n = cls_label.shape
    grid = (b * n) // _BLK
    labels = jnp.reshape(cls_label, (grid, 1, _BLK))
    out = pl.pallas_call(
        _body,
        grid=(grid,),
        in_specs=[pl.BlockSpec((1, 1, _BLK), lambda i: (i, 0, 0))],
        out_specs=pl.BlockSpec(memory_space=pltpu.MemorySpace.HBM),
        out_shape=jax.ShapeDtypeStruct((b, n, _C), jnp.int32),
        scratch_shapes=[
            pltpu.VMEM((_NBUF, _BLK, _C), jnp.int32),
            pltpu.SemaphoreType.DMA((_NBUF,)),
        ],
    )(labels)
    return out


# speedup vs baseline: 1.0495x; 1.0495x over previous
import jax
import jax.numpy as jnp
from jax.experimental import pallas as pl
from jax.experimental.pallas import tpu as pltpu

_C = 81
_BLK = 10000


def _onehot_body(lab_ref, out_ref):
    lf = lab_ref[0].astype(jnp.float32)
    ones = jnp.ones((1, _C), jnp.float32)
    lb = jax.lax.dot_general(
        lf, ones, (((0,), (0,)), ((), ())),
        preferred_element_type=jnp.float32)
    iota = jax.lax.broadcasted_iota(
        jnp.int32, (_BLK, _C), 1).astype(jnp.float32)
    out_ref[0] = jnp.where(lb == iota, 1,
                           jnp.where(lb == -iota, -1, 0)).astype(jnp.int32)


def kernel(cls_label):
    b, n = cls_label.shape
    grid = (b * n) // _BLK
    bpb = n // _BLK
    labels = jnp.reshape(cls_label, (grid, 1, _BLK))
    out = pl.pallas_call(
        _onehot_body,
        grid=(grid,),
        in_specs=[pl.BlockSpec((1, 1, _BLK), lambda i: (i, 0, 0))],
        out_specs=pl.BlockSpec(
            (1, _BLK, _C), lambda i: (i // bpb, i % bpb, 0)),
        out_shape=jax.ShapeDtypeStruct((b, n, _C), jnp.int32),
        compiler_params=pltpu.CompilerParams(
            dimension_semantics=("parallel",)),
    )(labels)
    return out


# speedup vs baseline: 1.2810x; 1.2206x over previous
import jax
import jax.numpy as jnp
from jax.experimental import pallas as pl
from jax.experimental.pallas import tpu as pltpu

_NUM_CLASSES = 81
_LANES = 128
_BLK = 20000


def _onehot_body(lab_ref, out_ref):
    lf = lab_ref[0].astype(jnp.float32)
    ones = jnp.ones((1, _LANES), jnp.float32)
    lb = jax.lax.dot_general(
        lf, ones, (((0,), (0,)), ((), ())),
        preferred_element_type=jnp.float32)
    iota = jax.lax.broadcasted_iota(
        jnp.int32, (_BLK, _LANES), 1).astype(jnp.float32)
    out_ref[0] = jnp.where(lb == iota, 1,
                           jnp.where(lb == -iota, -1, 0)).astype(jnp.int32)


def kernel(cls_label):
    b, n = cls_label.shape
    grid = (b * n) // _BLK
    blocks_per_batch = n // _BLK
    labels = jnp.reshape(cls_label, (grid, 1, _BLK))
    out = pl.pallas_call(
        _onehot_body,
        grid=(grid,),
        in_specs=[pl.BlockSpec((1, 1, _BLK), lambda i: (i, 0, 0))],
        out_specs=pl.BlockSpec(
            (1, _BLK, _LANES),
            lambda i: (i // blocks_per_batch, i % blocks_per_batch, 0)),
        out_shape=jax.ShapeDtypeStruct((b, n, _LANES), jnp.int32),
        compiler_params=pltpu.CompilerParams(
            dimension_semantics=("parallel",)),
    )(labels)
    return out[..., :_NUM_CLASSES]


# speedup vs baseline: 1.2816x; 1.0005x over previous
import jax
import jax.numpy as jnp
from jax.experimental import pallas as pl
from jax.experimental.pallas import tpu as pltpu

_NUM_CLASSES = 81
_LANES = 128
_BLK = 25000


def _onehot_body(lab_ref, out_ref):
    lf = lab_ref[0].astype(jnp.float32)
    ones = jnp.ones((1, _LANES), jnp.float32)
    lb = jax.lax.dot_general(
        lf, ones, (((0,), (0,)), ((), ())),
        preferred_element_type=jnp.float32)
    iota = jax.lax.broadcasted_iota(
        jnp.int32, (_BLK, _LANES), 1).astype(jnp.float32)
    out_ref[0] = jnp.where(lb == iota, 1,
                           jnp.where(lb == -iota, -1, 0)).astype(jnp.int32)


def kernel(cls_label):
    b, n = cls_label.shape
    grid = (b * n) // _BLK
    blocks_per_batch = n // _BLK
    labels = jnp.reshape(cls_label, (grid, 1, _BLK))
    out = pl.pallas_call(
        _onehot_body,
        grid=(grid,),
        in_specs=[pl.BlockSpec((1, 1, _BLK), lambda i: (i, 0, 0))],
        out_specs=pl.BlockSpec(
            (1, _BLK, _LANES),
            lambda i: (i // blocks_per_batch, i % blocks_per_batch, 0)),
        out_shape=jax.ShapeDtypeStruct((b, n, _LANES), jnp.int32),
        compiler_params=pltpu.CompilerParams(
            dimension_semantics=("parallel",)),
    )(labels)
    return out[..., :_NUM_CLASSES]
